# baseline (speedup 1.0000x reference)
import numpy as np

import jax
import jax.numpy as jnp
from jax import lax
from jax.experimental import pallas as pl
from jax.experimental.pallas import tpu as pltpu


def _fused_kernel(xt_ref, wa_ref, ba_ref, w15_ref, b15_ref, w2_ref, b2_ref, o_ref):
    h = jnp.dot(wa_ref[...], xt_ref[...], preferred_element_type=jnp.float32) + ba_ref[...]
    h = jnp.maximum(h, 0.0)
    h = jnp.dot(w15_ref[...], h, preferred_element_type=jnp.float32) + b15_ref[...]
    h = jnp.maximum(h, 0.0)
    h = jnp.dot(w2_ref[...], h, preferred_element_type=jnp.float32) + b2_ref[...]
    h = jnp.maximum(h, 0.0)
    m = jnp.max(h, axis=0, keepdims=True)
    e = jnp.exp(h - m)
    p = e / jnp.sum(e, axis=0, keepdims=True)
    o_ref[...] = p.T


def _toeplitz_sel(ks, kq):
    kt = ks + kq - 1
    s_y = np.repeat(np.arange(ks), ks)[:, None, None]
    s_x = np.tile(np.arange(ks), ks)[:, None, None]
    q_y = np.repeat(np.arange(kq), kq)[None, :, None]
    q_x = np.tile(np.arange(kq), kq)[None, :, None]
    t_y = np.repeat(np.arange(kt), kt)[None, None, :]
    t_x = np.tile(np.arange(kt), kt)[None, None, :]
    d = np.logical_and(t_y == q_y + s_y, t_x == q_x + s_x)
    return jnp.asarray(d.reshape(ks * ks, kq * kq * kt * kt), jnp.float32)


_D1 = _toeplitz_sel(7, 7)
_D2 = _toeplitz_sel(13, 3)

_P = np.zeros((784, 896), np.float32)
_P[np.arange(784), (np.arange(784) // 28) * 32 + np.arange(784) % 28] = 1.0
_P = jnp.asarray(_P)


def kernel(x, conv1_w, conv1_b, conv2_w, conv2_b, conv3_w, conv3_b,
           fc1_w, fc1_b, fc15_w, fc15_b, fc2_w, fc2_b):
    B = x.shape[0]

    t1 = jnp.dot(conv1_w.reshape(10, 49), _D1).reshape(490, 169)
    k12 = jnp.dot(conv2_w.reshape(20, 490), t1)
    b12 = conv2_b + jnp.einsum("ochw,c->o", conv2_w, conv1_b)
    t2 = jnp.dot(k12, _D2).reshape(180, 225)
    k123 = jnp.dot(conv3_w.reshape(30, 180), t2)
    b123 = conv3_b + jnp.einsum("ochw,c->o", conv3_w, b12)

    w1r = fc1_w.reshape(fc1_w.shape[0], 30, 14, 14)
    a4 = lax.conv_general_dilated(
        w1r, jnp.flip(k123.reshape(30, 15, 15), (1, 2)).reshape(1, 30, 15, 15),
        (1, 1), [(14, 14), (14, 14)],
        dimension_numbers=("NCHW", "OIHW", "NCHW"))
    wa = jnp.dot(a4.reshape(a4.shape[0], 784), _P)
    ba = (fc1_b + jnp.einsum("ochw,c->o", w1r, b123)).reshape(-1, 1)
    b15 = fc15_b.reshape(-1, 1)
    b2 = fc2_b.reshape(-1, 1)

    xt = jnp.concatenate(
        [jnp.transpose(x[:, 0], (1, 2, 0)),
         jnp.zeros((28, 4, B), jnp.float32)], axis=1).reshape(896, B)
    bn = B if B <= 1024 else 1024
    n_out = fc2_w.shape[0]
    return pl.pallas_call(
        _fused_kernel,
        out_shape=jax.ShapeDtypeStruct((B, n_out), jnp.float32),
        grid=(pl.cdiv(B, bn),),
        in_specs=[
            pl.BlockSpec((896, bn), lambda i: (0, i)),
            pl.BlockSpec(wa.shape, lambda i: (0, 0)),
            pl.BlockSpec(ba.shape, lambda i: (0, 0)),
            pl.BlockSpec(fc15_w.shape, lambda i: (0, 0)),
            pl.BlockSpec(b15.shape, lambda i: (0, 0)),
            pl.BlockSpec(fc2_w.shape, lambda i: (0, 0)),
            pl.BlockSpec(b2.shape, lambda i: (0, 0)),
        ],
        out_specs=pl.BlockSpec((bn, n_out), lambda i: (i, 0)),
        compiler_params=pltpu.CompilerParams(dimension_semantics=("parallel",)),
    )(xt, wa, ba, fc15_w, b15, fc2_w, b2)


# speedup vs baseline: 1.0537x; 1.0537x over previous
import numpy as np

import jax
import jax.numpy as jnp
from jax import lax
from jax.experimental import pallas as pl
from jax.experimental.pallas import tpu as pltpu


def _fused_kernel(xt_ref, wa_ref, ba_ref, w15_ref, b15_ref, w2_ref, b2_ref, o_ref):
    h = jnp.dot(wa_ref[...], xt_ref[...], preferred_element_type=jnp.float32) + ba_ref[...]
    h = jnp.maximum(h, 0.0)
    h = jnp.dot(w15_ref[...], h, preferred_element_type=jnp.float32) + b15_ref[...]
    h = jnp.maximum(h, 0.0)
    h = jnp.dot(w2_ref[...], h, preferred_element_type=jnp.float32) + b2_ref[...]
    h = jnp.maximum(h, 0.0)
    m = jnp.max(h, axis=0, keepdims=True)
    e = jnp.exp(h - m)
    p = e / jnp.sum(e, axis=0, keepdims=True)
    o_ref[...] = p.T


def _toeplitz_sel(ks, kq):
    kt = ks + kq - 1
    s_y = np.repeat(np.arange(ks), ks)[:, None, None]
    s_x = np.tile(np.arange(ks), ks)[:, None, None]
    q_y = np.repeat(np.arange(kq), kq)[None, :, None]
    q_x = np.tile(np.arange(kq), kq)[None, :, None]
    t_y = np.repeat(np.arange(kt), kt)[None, None, :]
    t_x = np.tile(np.arange(kt), kt)[None, None, :]
    d = np.logical_and(t_y == q_y + s_y, t_x == q_x + s_x)
    return jnp.asarray(d.reshape(ks * ks, kq * kq * kt * kt), jnp.float32)


_D1 = _toeplitz_sel(7, 7)
_D2 = _toeplitz_sel(13, 3)


def kernel(x, conv1_w, conv1_b, conv2_w, conv2_b, conv3_w, conv3_b,
           fc1_w, fc1_b, fc15_w, fc15_b, fc2_w, fc2_b):
    B = x.shape[0]

    t1 = jnp.dot(conv1_w.reshape(10, 49), _D1).reshape(490, 169)
    k12 = jnp.dot(conv2_w.reshape(20, 490), t1)
    b12 = conv2_b + jnp.einsum("ochw,c->o", conv2_w, conv1_b)
    t2 = jnp.dot(k12, _D2).reshape(180, 225)
    k123 = jnp.dot(conv3_w.reshape(30, 180), t2)
    b123 = conv3_b + jnp.einsum("ochw,c->o", conv3_w, b12)

    w1r = fc1_w.reshape(fc1_w.shape[0], 30, 14, 14)
    a4 = lax.conv_general_dilated(
        w1r, jnp.flip(k123.reshape(30, 15, 15), (1, 2)).reshape(1, 30, 15, 15),
        (1, 1), [(14, 14), (14, 14)],
        dimension_numbers=("NCHW", "OIHW", "NCHW"))
    wa = jnp.pad(a4.reshape(a4.shape[0], 28, 28),
                 ((0, 0), (0, 0), (0, 4))).reshape(a4.shape[0], 896)
    ba = (fc1_b + jnp.einsum("ochw,c->o", w1r, b123)).reshape(-1, 1)
    b15 = fc15_b.reshape(-1, 1)
    b2 = fc2_b.reshape(-1, 1)

    xt = jnp.pad(jnp.transpose(x[:, 0], (1, 2, 0)),
                 ((0, 0), (0, 4), (0, 0))).reshape(896, B)
    bn = B if B <= 2048 else 2048
    n_out = fc2_w.shape[0]
    return pl.pallas_call(
        _fused_kernel,
        out_shape=jax.ShapeDtypeStruct((B, n_out), jnp.float32),
        grid=(pl.cdiv(B, bn),),
        in_specs=[
            pl.BlockSpec((896, bn), lambda i: (0, i)),
            pl.BlockSpec(wa.shape, lambda i: (0, 0)),
            pl.BlockSpec(ba.shape, lambda i: (0, 0)),
            pl.BlockSpec(fc15_w.shape, lambda i: (0, 0)),
            pl.BlockSpec(b15.shape, lambda i: (0, 0)),
            pl.BlockSpec(fc2_w.shape, lambda i: (0, 0)),
            pl.BlockSpec(b2.shape, lambda i: (0, 0)),
        ],
        out_specs=pl.BlockSpec((bn, n_out), lambda i: (i, 0)),
        compiler_params=pltpu.CompilerParams(dimension_semantics=("parallel",)),
    )(xt, wa, ba, fc15_w, b15, fc2_w, b2)


# speedup vs baseline: 1.0582x; 1.0043x over previous
import numpy as np

import jax
import jax.numpy as jnp
from jax import lax
from jax.experimental import pallas as pl
from jax.experimental.pallas import tpu as pltpu


def _fused_kernel(xt_ref, wa_ref, ba_ref, w15_ref, b15_ref, w2_ref, b2_ref, o_ref):
    h = jnp.dot(wa_ref[...], xt_ref[...], preferred_element_type=jnp.float32) + ba_ref[...]
    h = jnp.maximum(h, 0.0)
    h = jnp.dot(w15_ref[...], h, preferred_element_type=jnp.float32) + b15_ref[...]
    h = jnp.maximum(h, 0.0)
    h = jnp.dot(w2_ref[...], h, preferred_element_type=jnp.float32) + b2_ref[...]
    h = jnp.maximum(h, 0.0)
    m = jnp.max(h, axis=0, keepdims=True)
    e = jnp.exp(h - m)
    p = e / jnp.sum(e, axis=0, keepdims=True)
    o_ref[...] = p.T


def _toeplitz_sel(ks, kq):
    kt = ks + kq - 1
    s_y = np.repeat(np.arange(ks), ks)[:, None, None]
    s_x = np.tile(np.arange(ks), ks)[:, None, None]
    q_y = np.repeat(np.arange(kq), kq)[None, :, None]
    q_x = np.tile(np.arange(kq), kq)[None, :, None]
    t_y = np.repeat(np.arange(kt), kt)[None, None, :]
    t_x = np.tile(np.arange(kt), kt)[None, None, :]
    d = np.logical_and(t_y == q_y + s_y, t_x == q_x + s_x)
    return d.reshape(ks * ks, kq * kq * kt * kt).astype(np.float32)


_D1 = _toeplitz_sel(7, 7)
_D2 = _toeplitz_sel(13, 3)


def kernel(x, conv1_w, conv1_b, conv2_w, conv2_b, conv3_w, conv3_b,
           fc1_w, fc1_b, fc15_w, fc15_b, fc2_w, fc2_b):
    B = x.shape[0]

    t1 = jnp.dot(conv1_w.reshape(10, 49), _D1).reshape(490, 169)
    k12 = jnp.dot(conv2_w.reshape(20, 490), t1)
    b12 = conv2_b + jnp.einsum("ochw,c->o", conv2_w, conv1_b)
    t2 = jnp.dot(k12, _D2).reshape(180, 225)
    k123 = jnp.dot(conv3_w.reshape(30, 180), t2)
    b123 = conv3_b + jnp.einsum("ochw,c->o", conv3_w, b12)

    w1r = fc1_w.reshape(fc1_w.shape[0], 30, 14, 14)
    a4 = lax.conv_general_dilated(
        w1r, jnp.flip(k123.reshape(30, 15, 15), (1, 2)).reshape(1, 30, 15, 15),
        (1, 1), [(14, 14), (14, 14)],
        dimension_numbers=("NCHW", "OIHW", "NCHW"))
    wa = jnp.pad(a4.reshape(a4.shape[0], 28, 28),
                 ((0, 0), (0, 0), (0, 4))).reshape(a4.shape[0], 896)
    ba = (fc1_b + jnp.einsum("ochw,c->o", w1r, b123)).reshape(-1, 1)
    b15 = fc15_b.reshape(-1, 1)
    b2 = fc2_b.reshape(-1, 1)

    xt = jnp.pad(jnp.transpose(x[:, 0], (1, 2, 0)),
                 ((0, 0), (0, 4), (0, 0))).reshape(896, B)
    bn = B if B <= 2048 else 2048
    n_out = fc2_w.shape[0]
    return pl.pallas_call(
        _fused_kernel,
        out_shape=jax.ShapeDtypeStruct((B, n_out), jnp.float32),
        grid=(pl.cdiv(B, bn),),
        in_specs=[
            pl.BlockSpec((896, bn), lambda i: (0, i)),
            pl.BlockSpec(wa.shape, lambda i: (0, 0)),
            pl.BlockSpec(ba.shape, lambda i: (0, 0)),
            pl.BlockSpec(fc15_w.shape, lambda i: (0, 0)),
            pl.BlockSpec(b15.shape, lambda i: (0, 0)),
            pl.BlockSpec(fc2_w.shape, lambda i: (0, 0)),
            pl.BlockSpec(b2.shape, lambda i: (0, 0)),
        ],
        out_specs=pl.BlockSpec((bn, n_out), lambda i: (i, 0)),
        compiler_params=pltpu.CompilerParams(dimension_semantics=("parallel",)),
    )(xt, wa, ba, fc15_w, b15, fc2_w, b2)
